# baseline (speedup 1.0000x reference)
import functools

import jax
import jax.numpy as jnp
from jax import lax
from jax.experimental import pallas as pl
from jax.experimental.pallas import tpu as pltpu
from jax.experimental.pallas import tpu_sc as plsc

NUM_CATEGORIES = 1000000
EMBEDDING_DIM = 64
BATCH = 16384
MAX_LABELS = 50

_INFO = plsc.get_sparse_core_info()
NC = _INFO.num_cores
NS = _INFO.num_subcores
NW = NC * NS
L = _INFO.num_lanes

C = 16
ROWS_PER_W = BATCH // NW
CHUNKS_PER_W = ROWS_PER_W // C
JGROUP = 10
NJ = MAX_LABELS // JGROUP
KD = EMBEDDING_DIM // L


def _body(xt_hbm, tbl_hbm, out_hbm, x_v, acc_v, o_v, sem):
    wid = lax.axis_index("s") * NC + lax.axis_index("c")
    zeros = jnp.zeros((L,), jnp.float32)
    ones = jnp.ones((L,), jnp.float32)

    def chunk_body(g, _):
        gg = wid * CHUNKS_PER_W + g
        base = gg * C
        pltpu.sync_copy(xt_hbm.at[gg], x_v)
        for r in range(C):
            for k in range(KD):
                acc_v[r, pl.ds(k * L, L)] = zeros

        def jgroup_body(jo, counts):
            descs = []
            for ji in range(JGROUP):
                j = jo * JGROUP + ji
                col = x_v[j]
                counts = counts + jnp.where(col != 0, ones, zeros)
                descs.append(
                    pltpu.async_copy(tbl_hbm.at[x_v.at[j]], acc_v, sem, add=True)
                )
            for d in descs:
                d.wait()
            return counts

        counts = lax.fori_loop(0, NJ, jgroup_body, zeros)
        recip = ones / (counts + jnp.float32(1e-8))
        for r in range(C):
            s = recip[r]
            for k in range(KD):
                o_v[r, pl.ds(k * L, L)] = acc_v[r, pl.ds(k * L, L)] * s
        pltpu.sync_copy(o_v, out_hbm.at[pl.ds(base, C), :])
        return 0

    lax.fori_loop(0, CHUNKS_PER_W, chunk_body, 0)


_TBLOCK = 4096


def _transpose_body(wt_ref, out_ref):
    eye = jnp.eye(EMBEDDING_DIM, dtype=jnp.float32)
    t = jax.lax.dot_general(
        wt_ref[...],
        eye,
        (((0,), (0,)), ((), ())),
        preferred_element_type=jnp.float32,
        precision=jax.lax.Precision.HIGHEST,
    )
    out_ref[:, 0:EMBEDDING_DIM] = t
    out_ref[:, EMBEDDING_DIM:128] = jnp.zeros(
        (_TBLOCK, 128 - EMBEDDING_DIM), jnp.float32
    )


def _transpose_pad(wt):
    grid = (NUM_CATEGORIES + _TBLOCK - 1) // _TBLOCK
    return pl.pallas_call(
        _transpose_body,
        grid=(grid,),
        in_specs=[pl.BlockSpec((EMBEDDING_DIM, _TBLOCK), lambda i: (0, i))],
        out_specs=pl.BlockSpec((_TBLOCK, 128), lambda i: (i, 0)),
        out_shape=jax.ShapeDtypeStruct((NUM_CATEGORIES, 128), jnp.float32),
    )(wt)


@jax.jit
def _encode(x, weight):
    mesh = plsc.VectorSubcoreMesh(core_axis_name="c", subcore_axis_name="s")
    f = pl.kernel(
        _body,
        out_type=jax.ShapeDtypeStruct((BATCH, EMBEDDING_DIM), jnp.float32),
        mesh=mesh,
        scratch_types=[
            pltpu.VMEM((MAX_LABELS, C), jnp.int32),
            pltpu.VMEM((C, 128), jnp.float32),
            pltpu.VMEM((C, EMBEDDING_DIM), jnp.float32),
            pltpu.SemaphoreType.DMA,
        ],
        compiler_params=pltpu.CompilerParams(use_tc_tiling_on_sc=True),
    )
    xt = x.reshape(BATCH // C, C, MAX_LABELS).transpose(0, 2, 1)
    wp = _transpose_pad(weight.T)
    return f(xt, wp)


def kernel(x, weight):
    return _encode(x, weight)


# speedup vs baseline: 1.2157x; 1.2157x over previous
import functools

import jax
import jax.numpy as jnp
from jax import lax
from jax.experimental import pallas as pl
from jax.experimental.pallas import tpu as pltpu
from jax.experimental.pallas import tpu_sc as plsc

NUM_CATEGORIES = 1000000
EMBEDDING_DIM = 64
BATCH = 16384
MAX_LABELS = 50

_INFO = plsc.get_sparse_core_info()
NC = _INFO.num_cores
NS = _INFO.num_subcores
NW = NC * NS
L = _INFO.num_lanes

C = 16
ROWS_PER_W = BATCH // NW
CHUNKS_PER_W = ROWS_PER_W // C
JGROUP = 10
NJ = MAX_LABELS // JGROUP
KD = EMBEDDING_DIM // L


def _body(xt_hbm, tbl_hbm, out_hbm, x_v, acc_v, o_v, sem):
    wid = lax.axis_index("s") * NC + lax.axis_index("c")
    zeros = jnp.zeros((L,), jnp.float32)
    ones = jnp.ones((L,), jnp.float32)

    def chunk_body(g, _):
        gg = wid * CHUNKS_PER_W + g
        base = gg * C
        pltpu.sync_copy(xt_hbm.at[gg], x_v)
        for r in range(C):
            for k in range(KD):
                acc_v[r, pl.ds(k * L, L)] = zeros

        def jgroup_body(jo, counts):
            descs = []
            for ji in range(JGROUP):
                j = jo * JGROUP + ji
                col = x_v[j]
                counts = counts + jnp.where(col != 0, ones, zeros)
                descs.append(
                    pltpu.async_copy(tbl_hbm.at[x_v.at[j]], acc_v, sem, add=True)
                )
            for d in descs:
                d.wait()
            return counts

        counts = lax.fori_loop(0, NJ, jgroup_body, zeros)
        recip = ones / (counts + jnp.float32(1e-8))
        for r in range(C):
            s = recip[r]
            for k in range(KD):
                o_v[r, pl.ds(k * L, L)] = acc_v[r, pl.ds(k * L, L)] * s
        pltpu.sync_copy(o_v, out_hbm.at[pl.ds(base, C), :])
        return 0

    lax.fori_loop(0, CHUNKS_PER_W, chunk_body, 0)


_TBLOCK = 4096


def _transpose_body(wt_ref, out_ref):
    t = jnp.transpose(wt_ref[...])
    out_ref[:, 0:EMBEDDING_DIM] = t


def _transpose_pad(wt):
    grid = (NUM_CATEGORIES + _TBLOCK - 1) // _TBLOCK
    return pl.pallas_call(
        _transpose_body,
        grid=(grid,),
        in_specs=[pl.BlockSpec((EMBEDDING_DIM, _TBLOCK), lambda i: (0, i))],
        out_specs=pl.BlockSpec((_TBLOCK, 128), lambda i: (i, 0)),
        out_shape=jax.ShapeDtypeStruct((NUM_CATEGORIES, 128), jnp.float32),
    )(wt)


@jax.jit
def _encode(x, weight):
    mesh = plsc.VectorSubcoreMesh(core_axis_name="c", subcore_axis_name="s")
    f = pl.kernel(
        _body,
        out_type=jax.ShapeDtypeStruct((BATCH, EMBEDDING_DIM), jnp.float32),
        mesh=mesh,
        scratch_types=[
            pltpu.VMEM((MAX_LABELS, C), jnp.int32),
            pltpu.VMEM((C, 128), jnp.float32),
            pltpu.VMEM((C, EMBEDDING_DIM), jnp.float32),
            pltpu.SemaphoreType.DMA,
        ],
        compiler_params=pltpu.CompilerParams(use_tc_tiling_on_sc=True),
    )
    xt = x.reshape(BATCH // C, C, MAX_LABELS).transpose(0, 2, 1)
    wp = _transpose_pad(weight.T)
    return f(xt, wp)


def kernel(x, weight):
    return _encode(x, weight)
